# baseline (speedup 1.0000x reference)
import functools

import jax
import jax.numpy as jnp
import numpy as np
from jax import lax
from jax.experimental import pallas as pl
from jax.experimental.pallas import tpu as pltpu
from jax.experimental.pallas import tpu_sc as plsc

_MASK = np.concatenate(
    [np.ones(25, dtype=bool), np.zeros(70, dtype=bool), np.ones(42, dtype=bool)]
)
_KEPT = np.nonzero(_MASK)[0].astype(np.int32)
_SRC_COLS = (3 * _KEPT[:, None] + np.arange(2)[None, :]).reshape(-1)

_FRAMES, _PEOPLE, _KP, _DIMS = 8192, 4, 137, 3
_ROWS = _FRAMES * _PEOPLE
_IN_W = _KP * _DIMS
_OUT_W = 2 * len(_KEPT)

_NW = 32
_ROWS_PER_W = _ROWS // _NW
_RC = 64
_CHUNKS = _ROWS_PER_W // _RC
_CHUNK_IN = _RC * _IN_W
_CHUNK_OUT = _RC * _OUT_W
_NVEC = _CHUNK_OUT // 16

_CHUNK_IDX = (
    np.arange(_RC, dtype=np.int32)[:, None] * _IN_W + _SRC_COLS[None, :]
).reshape(-1)


def _body(pose_hbm, cidx_hbm, out_hbm, cidx_v, in_v, out_v):
    wid = lax.axis_index("s") * 2 + lax.axis_index("c")
    pltpu.sync_copy(cidx_hbm, cidx_v)
    row0 = wid * _ROWS_PER_W

    def chunk(c, carry):
        in_off = pl.multiple_of((row0 + c * _RC) * _IN_W, _CHUNK_IN)
        out_off = pl.multiple_of((row0 + c * _RC) * _OUT_W, _CHUNK_OUT)
        pltpu.sync_copy(pose_hbm.at[pl.ds(in_off, _CHUNK_IN)], in_v)
        for t in range(_NVEC):
            idx = cidx_v[pl.ds(16 * t, 16)]
            out_v[pl.ds(16 * t, 16)] = plsc.load_gather(in_v, [idx])
        pltpu.sync_copy(out_v, out_hbm.at[pl.ds(out_off, _CHUNK_OUT)])
        return carry

    lax.fori_loop(0, _CHUNKS, chunk, 0)


_sc_filter = functools.partial(
    pl.kernel,
    mesh=plsc.VectorSubcoreMesh(core_axis_name="c", subcore_axis_name="s"),
    out_type=jax.ShapeDtypeStruct((_ROWS * _OUT_W,), jnp.float32),
    scratch_types=[
        pltpu.VMEM((_CHUNK_OUT,), jnp.int32),
        pltpu.VMEM((_CHUNK_IN,), jnp.float32),
        pltpu.VMEM((_CHUNK_OUT,), jnp.float32),
    ],
    compiler_params=pltpu.CompilerParams(needs_layout_passes=False),
)(_body)


def kernel(pose):
    flat = pose.reshape(-1)
    cidx = jnp.asarray(_CHUNK_IDX)
    out = _sc_filter(flat, cidx)
    return out.reshape(_FRAMES, _PEOPLE, len(_KEPT), 2)


# speedup vs baseline: 152.1481x; 152.1481x over previous
import functools

import jax
import jax.numpy as jnp
import numpy as np
from jax import lax
from jax.experimental import pallas as pl
from jax.experimental.pallas import tpu as pltpu
from jax.experimental.pallas import tpu_sc as plsc

_MASK = np.concatenate(
    [np.ones(25, dtype=bool), np.zeros(70, dtype=bool), np.ones(42, dtype=bool)]
)
_KEPT = np.nonzero(_MASK)[0].astype(np.int32)
_NKEPT = len(_KEPT)

_FRAMES, _PEOPLE, _KP, _DIMS = 8192, 4, 137, 3
_FT = _FRAMES // 128
_NW = 32

_SLABS = [(int(k), d) for k in _KEPT for d in (0, 1)]


def _body(in_hbm, out_hbm, slab_v):
    wid = lax.axis_index("s") * 2 + lax.axis_index("c")
    for i, (kp, d) in enumerate(_SLABS):
        kp_out = int(np.searchsorted(_KEPT, kp))

        @pl.when(wid == i % _NW)
        def _():
            s = kp * 3 + d
            pltpu.sync_copy(in_hbm.at[s], slab_v)
            for p in range(_PEOPLE):
                pltpu.sync_copy(slab_v.at[:, p, :], out_hbm.at[p, kp_out, :, d, :])


_sc_filter = functools.partial(
    pl.kernel,
    mesh=plsc.VectorSubcoreMesh(core_axis_name="c", subcore_axis_name="s"),
    out_type=jax.ShapeDtypeStruct((_PEOPLE, _NKEPT, _FT, 2, 128), jnp.float32),
    scratch_types=[
        pltpu.VMEM((_FT, _PEOPLE, 128), jnp.float32),
    ],
)(_body)


def kernel(pose):
    phys_in = jnp.transpose(
        pose.reshape(_FT, 128, _PEOPLE, _KP, _DIMS), (3, 4, 0, 2, 1)
    ).reshape(_KP * _DIMS, _FT, _PEOPLE, 128)
    out_phys = _sc_filter(phys_in)
    return jnp.transpose(out_phys, (2, 4, 0, 1, 3)).reshape(
        _FRAMES, _PEOPLE, _NKEPT, 2
    )


# speedup vs baseline: 223.7130x; 1.4704x over previous
import functools

import jax
import jax.numpy as jnp
import numpy as np
from jax import lax
from jax.experimental import pallas as pl
from jax.experimental.pallas import tpu as pltpu
from jax.experimental.pallas import tpu_sc as plsc

_MASK = np.concatenate(
    [np.ones(25, dtype=bool), np.zeros(70, dtype=bool), np.ones(42, dtype=bool)]
)
_KEPT = np.nonzero(_MASK)[0].astype(np.int32)
_NKEPT = len(_KEPT)

_FRAMES, _PEOPLE, _KP, _DIMS = 8192, 4, 137, 3
_FT = _FRAMES // 128
_NW = 32

_HT = _FT // 2
_NUNITS = _NKEPT * 2 * 2
_MAXJ = -(-_NUNITS // _NW)


def _body(in_hbm, out_hbm, b0, b1, ls0, ls1, ss0, ss1):
    wid = lax.axis_index("s") * 2 + lax.axis_index("c")
    bufs, lsems, ssems = (b0, b1), (ls0, ls1), (ss0, ss1)

    units = []
    for j in range(_MAXJ):
        u = wid + _NW * j
        i = u // 2
        h = u % 2
        k_out = i // 2
        d = i % 2
        kp = jnp.where(k_out >= 25, k_out + 70, k_out)
        s = 3 * kp + d
        b = j % 2
        ld = pltpu.make_async_copy(
            in_hbm.at[s, pl.ds(h * _HT, _HT)], bufs[b], lsems[b]
        )
        sts = [
            pltpu.make_async_copy(
                bufs[b].at[:, p, :],
                out_hbm.at[p, k_out, pl.ds(h * _HT, _HT), d, :],
                ssems[b],
            )
            for p in range(_PEOPLE)
        ]
        units.append((u, ld, sts))

    def guarded(j, fn):
        @pl.when(units[j][0] < _NUNITS)
        def _():
            fn()

    guarded(0, lambda: units[0][1].start())
    for j in range(_MAXJ):
        if j + 1 < _MAXJ:
            if j - 1 >= 0:
                guarded(j - 1, lambda j=j: [c.wait() for c in units[j - 1][2]])
            guarded(j + 1, lambda j=j: units[j + 1][1].start())
        guarded(j, lambda j=j: (units[j][1].wait(),
                                [c.start() for c in units[j][2]]))

    for j in (_MAXJ - 2, _MAXJ - 1):
        guarded(j, lambda j=j: [c.wait() for c in units[j][2]])


_sc_filter = functools.partial(
    pl.kernel,
    mesh=plsc.VectorSubcoreMesh(core_axis_name="c", subcore_axis_name="s"),
    out_type=jax.ShapeDtypeStruct((_PEOPLE, _NKEPT, _FT, 2, 128), jnp.float32),
    scratch_types=[
        pltpu.VMEM((_HT, _PEOPLE, 128), jnp.float32),
        pltpu.VMEM((_HT, _PEOPLE, 128), jnp.float32),
        pltpu.SemaphoreType.DMA,
        pltpu.SemaphoreType.DMA,
        pltpu.SemaphoreType.DMA,
        pltpu.SemaphoreType.DMA,
    ],
)(_body)


def kernel(pose):
    phys_in = jnp.transpose(
        pose.reshape(_FT, 128, _PEOPLE, _KP, _DIMS), (3, 4, 0, 2, 1)
    ).reshape(_KP * _DIMS, _FT, _PEOPLE, 128)
    out_phys = _sc_filter(phys_in)
    return jnp.transpose(out_phys, (2, 4, 0, 1, 3)).reshape(
        _FRAMES, _PEOPLE, _NKEPT, 2
    )


# speedup vs baseline: 230.7987x; 1.0317x over previous
import functools

import jax
import jax.numpy as jnp
import numpy as np
from jax import lax
from jax.experimental import pallas as pl
from jax.experimental.pallas import tpu as pltpu
from jax.experimental.pallas import tpu_sc as plsc

_MASK = np.concatenate(
    [np.ones(25, dtype=bool), np.zeros(70, dtype=bool), np.ones(42, dtype=bool)]
)
_KEPT = np.nonzero(_MASK)[0].astype(np.int32)
_NKEPT = len(_KEPT)

_FRAMES, _PEOPLE, _KP, _DIMS = 8192, 4, 137, 3
_FT = _FRAMES // 128
_NW = 32

_HT = _FT // 2
_NUNITS = _NKEPT * 2 * 2
_MAXJ = -(-_NUNITS // _NW)


_NBUF = 3


def _body(in_hbm, out_hbm, b0, b1, b2, ls0, ls1, ls2, ss0, ss1, ss2):
    wid = lax.axis_index("s") * 2 + lax.axis_index("c")
    bufs, lsems, ssems = (b0, b1, b2), (ls0, ls1, ls2), (ss0, ss1, ss2)

    units = []
    for j in range(_MAXJ):
        u = wid + _NW * j
        i = u // 2
        h = u % 2
        k_out = i // 2
        d = i % 2
        kp = jnp.where(k_out >= 25, k_out + 70, k_out)
        s = 3 * kp + d
        b = j % _NBUF
        ld = pltpu.make_async_copy(
            in_hbm.at[s, pl.ds(h * _HT, _HT)], bufs[b], lsems[b]
        )
        sts = [
            pltpu.make_async_copy(
                bufs[b].at[:, p, :],
                out_hbm.at[p, k_out, pl.ds(h * _HT, _HT), d, :],
                ssems[b],
            )
            for p in range(_PEOPLE)
        ]
        units.append((u, ld, sts))

    def guarded(j, fn):
        @pl.when(units[j][0] < _NUNITS)
        def _():
            fn()

    guarded(0, lambda: units[0][1].start())
    if _MAXJ > 1:
        guarded(1, lambda: units[1][1].start())
    for j in range(_MAXJ):
        if j + 2 < _MAXJ:
            if j - 1 >= 0:
                guarded(j - 1, lambda j=j: [c.wait() for c in units[j - 1][2]])
            guarded(j + 2, lambda j=j: units[j + 2][1].start())
        guarded(j, lambda j=j: (units[j][1].wait(),
                                [c.start() for c in units[j][2]]))

    for j in (_MAXJ - 3, _MAXJ - 2, _MAXJ - 1):
        if j >= 0:
            guarded(j, lambda j=j: [c.wait() for c in units[j][2]])


_sc_filter = functools.partial(
    pl.kernel,
    mesh=plsc.VectorSubcoreMesh(core_axis_name="c", subcore_axis_name="s"),
    out_type=jax.ShapeDtypeStruct((_PEOPLE, _NKEPT, _FT, 2, 128), jnp.float32),
    scratch_types=[
        pltpu.VMEM((_HT, _PEOPLE, 128), jnp.float32),
        pltpu.VMEM((_HT, _PEOPLE, 128), jnp.float32),
        pltpu.VMEM((_HT, _PEOPLE, 128), jnp.float32),
        pltpu.SemaphoreType.DMA,
        pltpu.SemaphoreType.DMA,
        pltpu.SemaphoreType.DMA,
        pltpu.SemaphoreType.DMA,
        pltpu.SemaphoreType.DMA,
        pltpu.SemaphoreType.DMA,
    ],
)(_body)


def kernel(pose):
    phys_in = jnp.transpose(
        pose.reshape(_FT, 128, _PEOPLE, _KP, _DIMS), (3, 4, 0, 2, 1)
    ).reshape(_KP * _DIMS, _FT, _PEOPLE, 128)
    out_phys = _sc_filter(phys_in)
    return jnp.transpose(out_phys, (2, 4, 0, 1, 3)).reshape(
        _FRAMES, _PEOPLE, _NKEPT, 2
    )
